# baseline (speedup 1.0000x reference)
import functools

import jax
import jax.numpy as jnp
from jax import lax
from jax.experimental import pallas as pl
from jax.experimental.pallas import tpu as pltpu
from jax.experimental.pallas import tpu_sc as plsc

BATCH = 16384
D = 64
L = 16
NC = 2
NS = 16
NW = NC * NS
CHUNK = BATCH // NW
SLAB = 256
NSLAB_MAIN = 3904 // NW
SENT = 0x7FFF0000
DUMMY = BATCH
STG = 128
NBUK = 128
CAP = 64

_mesh = plsc.VectorSubcoreMesh(core_axis_name="c", subcore_axis_name="s")
_params = pltpu.CompilerParams(needs_layout_passes=False)

_i16 = lambda v: jnp.full((L,), v, jnp.int32)


@functools.partial(
    pl.kernel,
    mesh=_mesh,
    compiler_params=_params,
    out_type=(jax.ShapeDtypeStruct((BATCH + L, 128), jnp.float32),
              jax.ShapeDtypeStruct((BATCH + L, 128), jnp.float32)),
    scratch_types=[
        pltpu.VMEM((2048,), jnp.int32),
        pltpu.VMEM((NBUK * CAP,), jnp.int32),
        pltpu.VMEM((NBUK * CAP,), jnp.int32),
        pltpu.VMEM((NBUK,), jnp.int32),
        pltpu.VMEM((BATCH + 2 * L,), jnp.int32),
        pltpu.VMEM((BATCH + 2 * L,), jnp.int32),
        pltpu.VMEM((2, D, SLAB), jnp.float32),
        pltpu.VMEM((STG, 128), jnp.float32),
        pltpu.VMEM((STG,), jnp.int32),
        pltpu.SMEM((NBUK,), jnp.int32),
        pltpu.SemaphoreType.DMA,
        pltpu.SemaphoreType.DMA,
        pltpu.SemaphoreType.DMA,
    ],
)
def _gather_relu_sc(uidx_hbm, iidx_hbm, utabT_hbm, itabT_hbm,
                    out_u_hbm, out_i_hbm,
                    chunk_v, bidx_v, bpos_v, fills_v, spix_v, sppo_v,
                    slab_v, stage_v, spost_v, fills_sm, semA, semB, semS):
    wid = lax.axis_index("s") * NC + lax.axis_index("c")
    iota16 = lax.iota(jnp.int32, L)
    c16 = [iota16 + _i16(16 * k) for k in range(D // L)]
    zero = jnp.zeros((L,), jnp.float32)

    for idx_hbm, tab_hbm, out_hbm in ((uidx_hbm, utabT_hbm, out_u_hbm),
                                      (iidx_hbm, itabT_hbm, out_i_hbm)):
        def pre_body(j, _):
            bidx_v[pl.ds(j * L, L)] = jnp.full((L,), SENT, jnp.int32)
            bpos_v[pl.ds(j * L, L)] = jnp.full((L,), DUMMY, jnp.int32)
            return 0

        lax.fori_loop(0, NBUK * CAP // L, pre_body, 0)
        for q in range(NBUK // L):
            fills_v[pl.ds(q * L, L)] = jnp.zeros((L,), jnp.int32)

        def chunk_body(ch, soff):
            pltpu.sync_copy(idx_hbm.at[pl.ds(ch * 2048, 2048)], chunk_v)

            def vec_body(j, off):
                v = chunk_v[pl.ds(j * L, L)]
                mine = ((v // SLAB) % NW) == wid
                t = v // (SLAB * NW)
                cnt, last = plsc.scan_count(t, mask=mine)
                fb = plsc.load_gather(fills_v, [t])
                slot_in = fb + cnt - 1
                ok = mine & (slot_in < CAP)
                dest = t * CAP + slot_in
                pos = ch * 2048 + j * L + iota16
                plsc.store_scatter(bidx_v, [dest], v, mask=ok)
                plsc.store_scatter(bpos_v, [dest], pos, mask=ok)
                plsc.addupdate_scatter(fills_v, [t], cnt, mask=last & mine)
                sp = mine & (slot_in >= CAP)
                plsc.store_compressed(spix_v.at[pl.ds(off, L)], v, mask=sp)
                plsc.store_compressed(sppo_v.at[pl.ds(off, L)], pos, mask=sp)
                return off + plsc.all_reduce_population_count(sp)[0]

            return lax.fori_loop(0, 2048 // L, vec_body, soff)

        spcount = lax.fori_loop(0, BATCH // 2048, chunk_body, jnp.int32(0))
        spix_v[pl.ds(spcount, L)] = jnp.full((L,), SENT, jnp.int32)
        sppo_v[pl.ds(spcount, L)] = jnp.full((L,), DUMMY, jnp.int32)
        sptrips = (spcount + L - 1) // L

        def mir_body(q, _):
            f16 = fills_v[pl.ds(q * L, L)]
            for e in range(L):
                fills_sm[q * L + e] = f16[e]
            return 0

        lax.fori_loop(0, NBUK // L, mir_body, 0)

        for q in range(STG // L):
            spost_v[pl.ds(q * L, L)] = jnp.full((L,), DUMMY, jnp.int32)

        def fetch(S, slot, sem):
            return pltpu.make_async_copy(
                tab_hbm.at[pl.ds(0, D),
                           pl.ds(pl.multiple_of(S * SLAB, 128), SLAB)],
                slab_v.at[slot], sem)

        def flush():
            pltpu.make_async_copy(stage_v, out_hbm.at[spost_v], semS).start()
            pltpu.make_async_copy(stage_v, out_hbm.at[spost_v], semS).wait()
            for q in range(STG // L):
                spost_v[pl.ds(q * L, L)] = jnp.full((L,), DUMMY, jnp.int32)

        def do_slab(S, tslot, slot, sc0):
            lo = S * SLAB
            cnt = fills_sm[tslot]

            def emit(sv, pv, sc):
                spost_v[pl.ds(sc, L)] = pv
                for e in range(L):
                    lane = _i16(0) + ((sv[e] - lo) & (SLAB - 1))
                    for k in range(D // L):
                        vk = plsc.load_gather(slab_v.at[slot], [c16[k], lane])
                        stage_v[sc + e, pl.ds(16 * k, L)] = jnp.maximum(vk, 0.0)
                sc = sc + L

                @pl.when(sc >= STG)
                def _():
                    flush()

                return jnp.where(sc >= STG, jnp.int32(0), sc)

            def proc(j2, sc):
                sv = bidx_v[pl.ds(tslot * CAP + j2 * L, L)]
                pv = bpos_v[pl.ds(tslot * CAP + j2 * L, L)]
                return emit(sv, pv, sc)

            sc1 = lax.fori_loop(0, (jnp.minimum(cnt, CAP) + L - 1) // L,
                                proc, sc0)

            def sproc(j3, sc):
                sv = spix_v[pl.ds(j3 * L, L)]
                pv = sppo_v[pl.ds(j3 * L, L)]
                valid = (sv >= lo) & (sv < lo + SLAB)
                pvm = jnp.where(valid, pv, jnp.int32(DUMMY))
                return emit(sv, pvm, sc)

            return lax.fori_loop(0, sptrips, sproc, sc1)

        fetch(wid, 0, semA).start()

        def slab_body(t, sc):
            s0 = wid + NW * (2 * t)
            s1 = wid + NW * (2 * t + 1)
            fetch(s1, 1, semB).start()
            fetch(s0, 0, semA).wait()
            sc = do_slab(s0, 2 * t, 0, sc)

            @pl.when(t + 1 < NSLAB_MAIN // 2)
            def _():
                fetch(s0 + 2 * NW, 0, semA).start()

            fetch(s1, 1, semB).wait()
            return do_slab(s1, 2 * t + 1, 1, sc)

        sc = lax.fori_loop(0, NSLAB_MAIN // 2, slab_body, jnp.int32(0))

        @pl.when(wid == 0)
        def _():
            pltpu.sync_copy(tab_hbm.at[pl.ds(0, D), pl.ds(3904 * SLAB, SLAB)],
                            slab_v.at[0])
            do_slab(jnp.int32(3904), NSLAB_MAIN, 0, sc)
            flush()

        @pl.when(wid == 1)
        def _():
            pltpu.sync_copy(tab_hbm.at[pl.ds(0, D), pl.ds(3905 * SLAB, SLAB)],
                            slab_v.at[0])
            do_slab(jnp.int32(3905), NSLAB_MAIN, 0, sc)
            flush()

        @pl.when(wid == 2)
        def _():
            toff = pl.multiple_of(jnp.int32(3906) * SLAB, 128)
            pltpu.sync_copy(tab_hbm.at[pl.ds(0, D), pl.ds(toff, 128)],
                            slab_v.at[0, pl.ds(0, D), pl.ds(0, 128)])
            do_slab(jnp.int32(3906), NSLAB_MAIN, 0, sc)
            flush()

        @pl.when(wid > 2)
        def _():
            flush()


@functools.partial(
    pl.kernel,
    mesh=_mesh,
    compiler_params=_params,
    out_type=jax.ShapeDtypeStruct((BATCH,), jnp.float32),
    scratch_types=[
        pltpu.VMEM((128, 128), jnp.float32),
        pltpu.VMEM((128, 128), jnp.float32),
        pltpu.VMEM((CHUNK,), jnp.float32),
        pltpu.SemaphoreType.DMA,
    ],
)
def _dot_head_sc(ru_hbm, ri_hbm, out_hbm, ub_v, ib_v, res_v, sem):
    wid = lax.axis_index("s") * NC + lax.axis_index("c")
    base = wid * CHUNK
    iota16 = lax.iota(jnp.int32, L)
    zero = jnp.zeros((L,), jnp.float32)

    def blk_body(b, _):
        off = base + b * 128
        cu = pltpu.async_copy(ru_hbm.at[pl.ds(off, 128)], ub_v, sem)
        ci = pltpu.async_copy(ri_hbm.at[pl.ds(off, 128)], ib_v, sem)
        cu.wait()
        ci.wait()

        def grp(g, _2):
            out16 = zero
            for e in range(L):
                acc = zero
                for k in range(D // L):
                    uv = ub_v[g * L + e, pl.ds(16 * k, L)]
                    iv = ib_v[g * L + e, pl.ds(16 * k, L)]
                    acc = acc + uv * iv
                out16 = jnp.where(iota16 == e, jnp.sum(acc), out16)
            res_v[pl.ds(b * 128 + g * L, L)] = 4.0 / (1.0 + jnp.exp(-out16)) + 1.0
            return 0

        lax.fori_loop(0, 128 // L, grp, 0)
        return 0

    lax.fori_loop(0, CHUNK // 128, blk_body, 0)
    pltpu.sync_copy(res_v, out_hbm.at[pl.ds(base, CHUNK)])


def kernel(user_indices, item_indices, user_table, item_table):
    ru, ri = _gather_relu_sc(user_indices.astype(jnp.int32),
                             item_indices.astype(jnp.int32),
                             user_table.T, item_table.T)
    return _dot_head_sc(ru, ri)


# speedup vs baseline: 2.2855x; 2.2855x over previous
import functools

import jax
import jax.numpy as jnp
from jax import lax
from jax.experimental import pallas as pl
from jax.experimental.pallas import tpu as pltpu
from jax.experimental.pallas import tpu_sc as plsc

BATCH = 16384
D = 64
L = 16
NC = 2
NS = 16
NW = NC * NS
CHUNK = BATCH // NW
SLAB = 512
NPAIRS = 30
SENT = 0x7FFF0000
DUMMY = BATCH
STG = 128
NBUK = 64
CAP = 64

_mesh = plsc.VectorSubcoreMesh(core_axis_name="c", subcore_axis_name="s")
_params = pltpu.CompilerParams(needs_layout_passes=False)

_i16 = lambda v: jnp.full((L,), v, jnp.int32)


@functools.partial(
    pl.kernel,
    mesh=_mesh,
    compiler_params=_params,
    out_type=(jax.ShapeDtypeStruct((BATCH + L, 128), jnp.float32),
              jax.ShapeDtypeStruct((BATCH + L, 128), jnp.float32)),
    scratch_types=[
        pltpu.VMEM((2048,), jnp.int32),
        pltpu.VMEM((NBUK * CAP,), jnp.int32),
        pltpu.VMEM((NBUK * CAP,), jnp.int32),
        pltpu.VMEM((NBUK,), jnp.int32),
        pltpu.VMEM((BATCH + 2 * L,), jnp.int32),
        pltpu.VMEM((BATCH + 2 * L,), jnp.int32),
        pltpu.VMEM((2, D, SLAB), jnp.float32),
        pltpu.VMEM((STG, 128), jnp.float32),
        pltpu.VMEM((STG,), jnp.int32),
        pltpu.SMEM((NBUK,), jnp.int32),
        pltpu.SemaphoreType.DMA,
        pltpu.SemaphoreType.DMA,
        pltpu.SemaphoreType.DMA,
    ],
)
def _gather_relu_sc(uidx_hbm, iidx_hbm, utabT_hbm, itabT_hbm,
                    out_u_hbm, out_i_hbm,
                    chunk_v, bidx_v, bpos_v, fills_v, spix_v, sppo_v,
                    slab_v, stage_v, spost_v, fills_sm, semA, semB, semS):
    wid = lax.axis_index("s") * NC + lax.axis_index("c")
    iota16 = lax.iota(jnp.int32, L)
    c16 = [iota16 + _i16(16 * k) for k in range(D // L)]
    zero = jnp.zeros((L,), jnp.float32)

    for idx_hbm, tab_hbm, out_hbm in ((uidx_hbm, utabT_hbm, out_u_hbm),
                                      (iidx_hbm, itabT_hbm, out_i_hbm)):
        def pre_body(j, _):
            bidx_v[pl.ds(j * L, L)] = jnp.full((L,), SENT, jnp.int32)
            bpos_v[pl.ds(j * L, L)] = jnp.full((L,), DUMMY, jnp.int32)
            return 0

        lax.fori_loop(0, NBUK * CAP // L, pre_body, 0)
        for q in range(NBUK // L):
            fills_v[pl.ds(q * L, L)] = jnp.zeros((L,), jnp.int32)

        def chunk_body(ch, soff):
            pltpu.sync_copy(idx_hbm.at[pl.ds(ch * 2048, 2048)], chunk_v)

            def vec_body(j, off):
                v = chunk_v[pl.ds(j * L, L)]
                mine = ((v // SLAB) % NW) == wid
                t = v // (SLAB * NW)
                cnt, last = plsc.scan_count(t, mask=mine)
                fb = plsc.load_gather(fills_v, [t])
                slot_in = fb + cnt - 1
                ok = mine & (slot_in < CAP)
                dest = t * CAP + slot_in
                pos = ch * 2048 + j * L + iota16
                plsc.store_scatter(bidx_v, [dest], v, mask=ok)
                plsc.store_scatter(bpos_v, [dest], pos, mask=ok)
                plsc.addupdate_scatter(fills_v, [t], cnt, mask=last & mine)
                sp = mine & (slot_in >= CAP)
                plsc.store_compressed(spix_v.at[pl.ds(off, L)], v, mask=sp)
                plsc.store_compressed(sppo_v.at[pl.ds(off, L)], pos, mask=sp)
                return off + plsc.all_reduce_population_count(sp)[0]

            return lax.fori_loop(0, 2048 // L, vec_body, soff)

        spcount = lax.fori_loop(0, BATCH // 2048, chunk_body, jnp.int32(0))
        spix_v[pl.ds(spcount, L)] = jnp.full((L,), SENT, jnp.int32)
        sppo_v[pl.ds(spcount, L)] = jnp.full((L,), DUMMY, jnp.int32)
        sptrips = (spcount + L - 1) // L

        def mir_body(q, _):
            f16 = fills_v[pl.ds(q * L, L)]
            for e in range(L):
                fills_sm[q * L + e] = f16[e]
            return 0

        lax.fori_loop(0, NBUK // L, mir_body, 0)

        for q in range(STG // L):
            spost_v[pl.ds(q * L, L)] = jnp.full((L,), DUMMY, jnp.int32)

        def fetch(S, slot, sem, width=SLAB):
            return [pltpu.make_async_copy(
                tab_hbm.at[pl.ds(8 * k, 8),
                           pl.ds(pl.multiple_of(S * SLAB, 128), width)],
                slab_v.at[slot, pl.ds(8 * k, 8), pl.ds(0, width)], sem)
                for k in range(D // 8)]

        def start_all(cs):
            for c in cs:
                c.start()

        def wait_all(cs):
            for c in cs:
                c.wait()

        def flush():
            pltpu.make_async_copy(stage_v, out_hbm.at[spost_v], semS).start()
            pltpu.make_async_copy(stage_v, out_hbm.at[spost_v], semS).wait()
            for q in range(STG // L):
                spost_v[pl.ds(q * L, L)] = jnp.full((L,), DUMMY, jnp.int32)

        def do_slab(S, tslot, slot, sc0):
            lo = S * SLAB
            cnt = fills_sm[tslot]

            def emit(sv, pv, sc):
                spost_v[pl.ds(sc, L)] = pv
                for e in range(L):
                    lane = _i16(0) + ((sv[e] - lo) & (SLAB - 1))
                    for k in range(D // L):
                        vk = plsc.load_gather(slab_v.at[slot], [c16[k], lane])
                        stage_v[sc + e, pl.ds(16 * k, L)] = jnp.maximum(vk, 0.0)
                sc = sc + L

                @pl.when(sc >= STG)
                def _():
                    flush()

                return jnp.where(sc >= STG, jnp.int32(0), sc)

            def proc(j2, sc):
                sv = bidx_v[pl.ds(tslot * CAP + j2 * L, L)]
                pv = bpos_v[pl.ds(tslot * CAP + j2 * L, L)]
                return emit(sv, pv, sc)

            sc1 = lax.fori_loop(0, (jnp.minimum(cnt, CAP) + L - 1) // L,
                                proc, sc0)

            def sproc(j3, sc):
                sv = spix_v[pl.ds(j3 * L, L)]
                pv = sppo_v[pl.ds(j3 * L, L)]
                valid = (sv >= lo) & (sv < lo + SLAB)
                pvm = jnp.where(valid, pv, jnp.int32(DUMMY))
                return emit(sv, pvm, sc)

            return lax.fori_loop(0, sptrips, sproc, sc1)

        start_all(fetch(wid, 0, semA))

        def slab_body(t, sc):
            s0 = wid + NW * (2 * t)
            s1 = wid + NW * (2 * t + 1)
            start_all(fetch(s1, 1, semB))
            wait_all(fetch(s0, 0, semA))
            sc = do_slab(s0, 2 * t, 0, sc)
            start_all(fetch(s0 + 2 * NW, 0, semA))
            wait_all(fetch(s1, 1, semB))
            return do_slab(s1, 2 * t + 1, 1, sc)

        sc = lax.fori_loop(0, NPAIRS, slab_body, jnp.int32(0))

        wait_all(fetch(wid + NW * 2 * NPAIRS, 0, semA))
        sc = do_slab(wid + NW * 2 * NPAIRS, 2 * NPAIRS, 0, sc)

        @pl.when(wid == 0)
        def _():
            cs = fetch(jnp.int32(1952), 1, semB)
            start_all(cs)
            wait_all(cs)
            do_slab(jnp.int32(1952), 2 * NPAIRS + 1, 1, sc)
            flush()

        @pl.when(wid == 1)
        def _():
            cs = fetch(jnp.int32(1953), 1, semB, width=128)
            start_all(cs)
            wait_all(cs)
            do_slab(jnp.int32(1953), 2 * NPAIRS + 1, 1, sc)
            flush()

        @pl.when(wid > 1)
        def _():
            flush()


@functools.partial(
    pl.kernel,
    mesh=_mesh,
    compiler_params=_params,
    out_type=jax.ShapeDtypeStruct((BATCH,), jnp.float32),
    scratch_types=[
        pltpu.VMEM((128, 128), jnp.float32),
        pltpu.VMEM((128, 128), jnp.float32),
        pltpu.VMEM((CHUNK,), jnp.float32),
        pltpu.SemaphoreType.DMA,
    ],
)
def _dot_head_sc(ru_hbm, ri_hbm, out_hbm, ub_v, ib_v, res_v, sem):
    wid = lax.axis_index("s") * NC + lax.axis_index("c")
    base = wid * CHUNK
    iota16 = lax.iota(jnp.int32, L)
    zero = jnp.zeros((L,), jnp.float32)

    def blk_body(b, _):
        off = base + b * 128
        cu = pltpu.async_copy(ru_hbm.at[pl.ds(off, 128)], ub_v, sem)
        ci = pltpu.async_copy(ri_hbm.at[pl.ds(off, 128)], ib_v, sem)
        cu.wait()
        ci.wait()

        def grp(g, _2):
            out16 = zero
            for e in range(L):
                acc = zero
                for k in range(D // L):
                    uv = ub_v[g * L + e, pl.ds(16 * k, L)]
                    iv = ib_v[g * L + e, pl.ds(16 * k, L)]
                    acc = acc + uv * iv
                out16 = jnp.where(iota16 == e, jnp.sum(acc), out16)
            res_v[pl.ds(b * 128 + g * L, L)] = 4.0 / (1.0 + jnp.exp(-out16)) + 1.0
            return 0

        lax.fori_loop(0, 128 // L, grp, 0)
        return 0

    lax.fori_loop(0, CHUNK // 128, blk_body, 0)
    pltpu.sync_copy(res_v, out_hbm.at[pl.ds(base, CHUNK)])


def kernel(user_indices, item_indices, user_table, item_table):
    ru, ri = _gather_relu_sc(user_indices.astype(jnp.int32),
                             item_indices.astype(jnp.int32),
                             user_table.T, item_table.T)
    return _dot_head_sc(ru, ri)


# speedup vs baseline: 8.9506x; 3.9162x over previous
import functools

import jax
import jax.numpy as jnp
from jax import lax
from jax.experimental import pallas as pl
from jax.experimental.pallas import tpu as pltpu
from jax.experimental.pallas import tpu_sc as plsc

BATCH = 16384
D = 64
L = 16
NC = 2
NS = 16
NW = NC * NS
CHUNK = BATCH // NW
GUSERS = 8
NIDX = 128

_mesh = plsc.VectorSubcoreMesh(core_axis_name="c", subcore_axis_name="s")


@functools.partial(
    pl.kernel,
    mesh=_mesh,
    compiler_params=pltpu.CompilerParams(needs_layout_passes=False),
    out_type=jax.ShapeDtypeStruct((BATCH,), jnp.float32),
    scratch_types=[
        pltpu.VMEM((CHUNK // NIDX, NIDX), jnp.int32),
        pltpu.VMEM((CHUNK // NIDX, NIDX), jnp.int32),
        pltpu.VMEM((2, 2, D, 128), jnp.float32),
        pltpu.VMEM((2, 2, D, 128), jnp.float32),
        pltpu.VMEM((CHUNK,), jnp.float32),
        pltpu.SemaphoreType.DMA,
        pltpu.SemaphoreType.DMA,
    ],
)
def _rec_sc(uidx_hbm, iidx_hbm, utabT_hbm, itabT_hbm, out_hbm,
            uidx_v, iidx_v, ubuf_v, ibuf_v, res_v, sem0, sem1):
    wid = lax.axis_index("s") * NC + lax.axis_index("c")
    base = wid * CHUNK

    for j in range(CHUNK // NIDX):
        pltpu.sync_copy(uidx_hbm.at[pl.ds(base + j * NIDX, NIDX)], uidx_v.at[j])
        pltpu.sync_copy(iidx_hbm.at[pl.ds(base + j * NIDX, NIDX)], iidx_v.at[j])

    iota16 = lax.iota(jnp.int32, L)
    c16 = [jnp.full((L,), 0, jnp.int32) + iota16 + 16 * k for k in range(D // L)]
    zero = jnp.zeros((L,), jnp.float32)

    sems = [sem0, sem1]

    def grp_body(g, _):
        ridx_u = uidx_v[g // (NIDX // L), pl.ds((g % (NIDX // L)) * L, L)]
        ridx_i = iidx_v[g // (NIDX // L), pl.ds((g % (NIDX // L)) * L, L)]

        def issue(t):
            s = t % 2
            cs = []
            for p in range(2):
                r_u = ridx_u[t * 2 + p]
                r_i = ridx_i[t * 2 + p]
                w_u = pl.multiple_of((r_u // 128) * 128, 128)
                w_i = pl.multiple_of((r_i // 128) * 128, 128)
                for k in range(D // 8):
                    cs.append(pltpu.async_copy(
                        utabT_hbm.at[pl.ds(8 * k, 8), pl.ds(w_u, 128)],
                        ubuf_v.at[s, p, pl.ds(8 * k, 8), pl.ds(0, 128)],
                        sems[s]))
                    cs.append(pltpu.async_copy(
                        itabT_hbm.at[pl.ds(8 * k, 8), pl.ds(w_i, 128)],
                        ibuf_v.at[s, p, pl.ds(8 * k, 8), pl.ds(0, 128)],
                        sems[s]))
            return cs

        out16 = zero
        pend = {0: issue(0)}
        for t in range(8):
            s = t % 2
            if t < 7:
                pend[t + 1] = issue(t + 1)
            for c in pend.pop(t):
                c.wait()
            for p in range(2):
                o_u = jnp.full((L,), 0, jnp.int32) + (ridx_u[t * 2 + p] % 128)
                o_i = jnp.full((L,), 0, jnp.int32) + (ridx_i[t * 2 + p] % 128)
                acc = zero
                for k in range(D // L):
                    uvec = plsc.load_gather(ubuf_v.at[s, p], [c16[k], o_u])
                    ivec = plsc.load_gather(ibuf_v.at[s, p], [c16[k], o_i])
                    acc = acc + jnp.maximum(uvec, 0.0) * jnp.maximum(ivec, 0.0)
                out16 = jnp.where(iota16 == (t * 2 + p), jnp.sum(acc), out16)
        res_v[pl.ds(g * L, L)] = 4.0 / (1.0 + jnp.exp(-out16)) + 1.0
        return 0

    lax.fori_loop(0, CHUNK // L, grp_body, 0)

    pltpu.sync_copy(res_v, out_hbm.at[pl.ds(base, CHUNK)])


def kernel(user_indices, item_indices, user_table, item_table):
    return _rec_sc(user_indices.astype(jnp.int32),
                   item_indices.astype(jnp.int32),
                   user_table.T, item_table.T)


# speedup vs baseline: 9.0713x; 1.0135x over previous
import functools

import jax
import jax.numpy as jnp
from jax import lax
from jax.experimental import pallas as pl
from jax.experimental.pallas import tpu as pltpu
from jax.experimental.pallas import tpu_sc as plsc

BATCH = 16384
D = 64
L = 16
NC = 2
NS = 16
NW = NC * NS
CHUNK = BATCH // NW
GUSERS = 8
NIDX = 128

_mesh = plsc.VectorSubcoreMesh(core_axis_name="c", subcore_axis_name="s")


@functools.partial(
    pl.kernel,
    mesh=_mesh,
    compiler_params=pltpu.CompilerParams(needs_layout_passes=False),
    out_type=jax.ShapeDtypeStruct((BATCH,), jnp.float32),
    scratch_types=[
        pltpu.VMEM((CHUNK // NIDX, NIDX), jnp.int32),
        pltpu.VMEM((CHUNK // NIDX, NIDX), jnp.int32),
        pltpu.VMEM((2, 2, D, 128), jnp.float32),
        pltpu.VMEM((2, 2, D, 128), jnp.float32),
        pltpu.VMEM((CHUNK,), jnp.float32),
        pltpu.SemaphoreType.DMA,
        pltpu.SemaphoreType.DMA,
    ],
)
def _rec_sc(uidx_hbm, iidx_hbm, utabT_hbm, itabT_hbm, out_hbm,
            uidx_v, iidx_v, ubuf_v, ibuf_v, res_v, sem0, sem1):
    wid = lax.axis_index("s") * NC + lax.axis_index("c")
    base = wid * CHUNK

    for j in range(CHUNK // NIDX):
        pltpu.sync_copy(uidx_hbm.at[pl.ds(base + j * NIDX, NIDX)], uidx_v.at[j])
        pltpu.sync_copy(iidx_hbm.at[pl.ds(base + j * NIDX, NIDX)], iidx_v.at[j])

    iota16 = lax.iota(jnp.int32, L)
    c16 = [jnp.full((L,), 0, jnp.int32) + iota16 + 16 * k for k in range(D // L)]
    zero = jnp.zeros((L,), jnp.float32)

    sems = [sem0, sem1]

    def grp_body(g, _):
        ridx_u = uidx_v[g // (NIDX // L), pl.ds((g % (NIDX // L)) * L, L)]
        ridx_i = iidx_v[g // (NIDX // L), pl.ds((g % (NIDX // L)) * L, L)]

        def issue(t):
            s = t % 2
            cs = []
            for p in range(2):
                r_u = ridx_u[t * 2 + p]
                r_i = ridx_i[t * 2 + p]
                cs.append(pltpu.async_copy(
                    utabT_hbm.at[pl.ds(0, D),
                                 pl.ds(pl.multiple_of((r_u // 128) * 128, 128), 128)],
                    ubuf_v.at[s, p], sems[s]))
                cs.append(pltpu.async_copy(
                    itabT_hbm.at[pl.ds(0, D),
                                 pl.ds(pl.multiple_of((r_i // 128) * 128, 128), 128)],
                    ibuf_v.at[s, p], sems[s]))
            return cs

        out16 = zero
        pend = {0: issue(0)}
        for t in range(8):
            s = t % 2
            if t < 7:
                pend[t + 1] = issue(t + 1)
            for c in pend.pop(t):
                c.wait()
            for p in range(2):
                o_u = jnp.full((L,), 0, jnp.int32) + (ridx_u[t * 2 + p] % 128)
                o_i = jnp.full((L,), 0, jnp.int32) + (ridx_i[t * 2 + p] % 128)
                acc = zero
                for k in range(D // L):
                    uvec = plsc.load_gather(ubuf_v.at[s, p], [c16[k], o_u])
                    ivec = plsc.load_gather(ibuf_v.at[s, p], [c16[k], o_i])
                    acc = acc + jnp.maximum(uvec, 0.0) * jnp.maximum(ivec, 0.0)
                out16 = jnp.where(iota16 == (t * 2 + p), jnp.sum(acc), out16)
        res_v[pl.ds(g * L, L)] = 4.0 / (1.0 + jnp.exp(-out16)) + 1.0
        return 0

    lax.fori_loop(0, CHUNK // L, grp_body, 0)

    pltpu.sync_copy(res_v, out_hbm.at[pl.ds(base, CHUNK)])


def kernel(user_indices, item_indices, user_table, item_table):
    return _rec_sc(user_indices.astype(jnp.int32),
                   item_indices.astype(jnp.int32),
                   user_table.T, item_table.T)


# speedup vs baseline: 9.1425x; 1.0078x over previous
import functools

import jax
import jax.numpy as jnp
from jax import lax
from jax.experimental import pallas as pl
from jax.experimental.pallas import tpu as pltpu
from jax.experimental.pallas import tpu_sc as plsc

BATCH = 16384
D = 64
L = 16
NC = 2
NS = 16
NW = NC * NS
CHUNK = BATCH // NW
GUSERS = 8
NIDX = 128

_mesh = plsc.VectorSubcoreMesh(core_axis_name="c", subcore_axis_name="s")


@functools.partial(
    pl.kernel,
    mesh=_mesh,
    compiler_params=pltpu.CompilerParams(needs_layout_passes=False),
    out_type=jax.ShapeDtypeStruct((BATCH,), jnp.float32),
    scratch_types=[
        pltpu.VMEM((CHUNK // NIDX, NIDX), jnp.int32),
        pltpu.VMEM((CHUNK // NIDX, NIDX), jnp.int32),
        pltpu.VMEM((4, D, 128), jnp.float32),
        pltpu.VMEM((4, D, 128), jnp.float32),
        pltpu.VMEM((CHUNK,), jnp.float32),
        pltpu.SemaphoreType.DMA,
    ],
)
def _rec_sc(uidx_hbm, iidx_hbm, utabT_hbm, itabT_hbm, out_hbm,
            uidx_v, iidx_v, ubuf_v, ibuf_v, res_v, sem):
    wid = lax.axis_index("s") * NC + lax.axis_index("c")
    base = wid * CHUNK

    for j in range(CHUNK // NIDX):
        pltpu.sync_copy(uidx_hbm.at[pl.ds(base + j * NIDX, NIDX)], uidx_v.at[j])
        pltpu.sync_copy(iidx_hbm.at[pl.ds(base + j * NIDX, NIDX)], iidx_v.at[j])

    iota16 = lax.iota(jnp.int32, L)
    c16 = [jnp.full((L,), 0, jnp.int32) + iota16 + 16 * k for k in range(D // L)]
    zero = jnp.zeros((L,), jnp.float32)

    def grp_body(g, _):
        ridx_u = uidx_v[g // (NIDX // L), pl.ds((g % (NIDX // L)) * L, L)]
        ridx_i = iidx_v[g // (NIDX // L), pl.ds((g % (NIDX // L)) * L, L)]
        out16 = zero
        for t in range(4):
            copies = []
            for p in range(4):
                r_u = ridx_u[t * 4 + p]
                r_i = ridx_i[t * 4 + p]
                copies.append(pltpu.async_copy(
                    utabT_hbm.at[pl.ds(0, D),
                                 pl.ds(pl.multiple_of((r_u // 128) * 128, 128), 128)],
                    ubuf_v.at[p], sem))
                copies.append(pltpu.async_copy(
                    itabT_hbm.at[pl.ds(0, D),
                                 pl.ds(pl.multiple_of((r_i // 128) * 128, 128), 128)],
                    ibuf_v.at[p], sem))
            for c in copies:
                c.wait()

            for p in range(4):
                o_u = jnp.full((L,), 0, jnp.int32) + (ridx_u[t * 4 + p] % 128)
                o_i = jnp.full((L,), 0, jnp.int32) + (ridx_i[t * 4 + p] % 128)
                acc = zero
                for k in range(D // L):
                    uvec = plsc.load_gather(ubuf_v.at[p], [c16[k], o_u])
                    ivec = plsc.load_gather(ibuf_v.at[p], [c16[k], o_i])
                    acc = acc + jnp.maximum(uvec, 0.0) * jnp.maximum(ivec, 0.0)
                out16 = jnp.where(iota16 == (t * 4 + p), jnp.sum(acc), out16)
        res_v[pl.ds(g * L, L)] = 4.0 / (1.0 + jnp.exp(-out16)) + 1.0
        return 0

    lax.fori_loop(0, CHUNK // L, grp_body, 0)

    pltpu.sync_copy(res_v, out_hbm.at[pl.ds(base, CHUNK)])


def kernel(user_indices, item_indices, user_table, item_table):
    return _rec_sc(user_indices.astype(jnp.int32),
                   item_indices.astype(jnp.int32),
                   user_table.T, item_table.T)
